# Initial kernel scaffold; baseline (speedup 1.0000x reference)
import jax
import jax.numpy as jnp
from jax.experimental import pallas as pl


def kernel(nodes, edges, senders, receivers, W_e1, b_e1, scale_e, offset_e, W_e2, b_e2, W_n1, b_n1, scale_n, offset_n, W_n2, b_n2):
    raise NotImplementedError("write your pallas kernel here")


# speedup vs baseline: 3.5491x; 3.5491x over previous
import functools

import jax
import jax.numpy as jnp
from jax import lax
from jax.experimental import pallas as pl
from jax.experimental.pallas import tpu as pltpu
from jax.experimental.pallas import tpu_sc as plsc

N = 10000
E = 320000
D = 128
DE = 4

NC = 2
NS = 16
NW = NC * NS
EW = E // NW
EB = 80
NBATCH = EW // EB
RPS = N // NS
CPAD = 640
NCNT = NS * CPAD
ZR = 25


def _prep_body(nodes_ref, ws_ref, wr_ref, ps_ref, pr_ref):
    x = nodes_ref[...]
    ps_ref[...] = jnp.dot(x, ws_ref[...], preferred_element_type=jnp.float32)
    pr_ref[...] = jnp.dot(x, wr_ref[...], preferred_element_type=jnp.float32)


def _edgebase_body(edges_ref, we_ref, b_ref, out_ref):
    out_ref[...] = (jnp.dot(edges_ref[...], we_ref[...],
                            preferred_element_type=jnp.float32)
                    + b_ref[...])


def _sc_body(ps_hbm, pr_hbm, eb_hbm, snd_hbm, rcv_hbm,
             acc_out, cnt_out,
             sidx, ridx, psv, prv, ebv, gv, onesv, zbuf, zcnt,
             acc_sh, cnt_sh, sem1, sem2):
    c = lax.axis_index("c")
    s = lax.axis_index("s")
    wid = s * NC + c

    zeros16 = jnp.zeros((16,), jnp.float32)
    ones16 = jnp.ones((16,), jnp.float32)

    def zfill_row(i, carry):
        for k in range(D // 16):
            zbuf[i, 16 * k:16 * (k + 1)] = zeros16
        return carry

    lax.fori_loop(0, ZR, zfill_row, 0)
    for k in range(CPAD // 16):
        zcnt[16 * k:16 * (k + 1)] = zeros16
    for k in range(EB // 16):
        onesv[16 * k:16 * (k + 1)] = ones16

    for i in range(RPS // ZR):
        pltpu.sync_copy(zbuf, acc_sh.at[pl.ds(s * RPS + i * ZR, ZR), :])
    pltpu.sync_copy(zcnt, cnt_sh.at[pl.ds(s * CPAD, CPAD)])
    plsc.subcore_barrier()

    base0 = wid * EW

    def batch(t, carry):
        base = base0 + t * EB
        pltpu.sync_copy(snd_hbm.at[pl.ds(base, EB)], sidx)
        pltpu.sync_copy(rcv_hbm.at[pl.ds(base, EB)], ridx)
        cp1 = pltpu.async_copy(ps_hbm.at[sidx], psv, sem1)
        cp2 = pltpu.async_copy(pr_hbm.at[ridx], prv, sem2)
        pltpu.sync_copy(eb_hbm.at[pl.ds(base, EB)], ebv)
        cp1.wait()
        cp2.wait()

        def edge(j, inner):
            r = []
            for k in range(D // 16):
                h = (psv[j, 16 * k:16 * (k + 1)]
                     + prv[j, 16 * k:16 * (k + 1)]
                     + ebv[j, 16 * k:16 * (k + 1)])
                r.append(jnp.maximum(h, 0.0))
            tot = ((r[0] + r[1]) + (r[2] + r[3])) + ((r[4] + r[5]) + (r[6] + r[7]))
            sq = [v * v for v in r]
            tot2 = ((sq[0] + sq[1]) + (sq[2] + sq[3])) + ((sq[4] + sq[5]) + (sq[6] + sq[7]))
            i16 = lax.iota(jnp.int32, 16)
            for kk in (8, 4, 2, 1):
                tot = tot + jnp.take(tot, i16 ^ kk)
                tot2 = tot2 + jnp.take(tot2, i16 ^ kk)
            muv = tot * (1.0 / D)
            xv = tot2 * (1.0 / D) - muv * muv + 1e-5
            y = lax.bitcast_convert_type(
                jnp.int32(0x5F3759DF) - (lax.bitcast_convert_type(xv, jnp.int32) >> 1),
                jnp.float32)
            for _ in range(3):
                y = y * (1.5 - 0.5 * xv * y * y)
            for k in range(D // 16):
                gv[j, 16 * k:16 * (k + 1)] = (r[k] - muv) * y
            return inner

        lax.fori_loop(0, EB, edge, 0)
        pltpu.sync_copy(gv, acc_sh.at[ridx], add=True)
        pltpu.sync_copy(onesv, cnt_sh.at[ridx], add=True)
        return carry

    lax.fori_loop(0, NBATCH, batch, 0)
    plsc.subcore_barrier()

    @pl.when(s == 0)
    def _copy_out():
        pltpu.sync_copy(acc_sh, acc_out.at[c])
        pltpu.sync_copy(cnt_sh, cnt_out.at[c])


def _post_body(acc2_ref, cnt2_ref, nodes_ref,
               scale_e_ref, offset_e_ref, we2_ref, be2_ref,
               wn1a_ref, wn1b_ref, bn1_ref, scale_n_ref, offset_n_ref,
               wn2_ref, bn2_ref, out_ref):
    acc = acc2_ref[0] + acc2_ref[1]
    cnt = cnt2_ref[0] + cnt2_ref[1]
    w2s = scale_e_ref[...].reshape(D, 1) * we2_ref[...]
    cvec = (jnp.dot(offset_e_ref[...].reshape(1, D), we2_ref[...],
                    preferred_element_type=jnp.float32)
            + be2_ref[...].reshape(1, D))
    agg = (jnp.dot(acc, w2s, preferred_element_type=jnp.float32)
           + cnt * cvec)
    g = (jnp.dot(nodes_ref[...], wn1a_ref[...], preferred_element_type=jnp.float32)
         + jnp.dot(agg, wn1b_ref[...], preferred_element_type=jnp.float32)
         + bn1_ref[...])
    g = jnp.maximum(g, 0.0)
    mu = jnp.mean(g, axis=-1, keepdims=True)
    var = jnp.mean(g * g, axis=-1, keepdims=True) - mu * mu
    y = (g - mu) * lax.rsqrt(var + 1e-5) * scale_n_ref[...] + offset_n_ref[...]
    out_ref[...] = (jnp.dot(y, wn2_ref[...], preferred_element_type=jnp.float32)
                    + bn2_ref[...])


def kernel(nodes, edges, senders, receivers,
           W_e1, b_e1, scale_e, offset_e, W_e2, b_e2,
           W_n1, b_n1, scale_n, offset_n, W_n2, b_n2):
    w_edge = W_e1[:DE]
    w_s = W_e1[DE:DE + D]
    w_r = W_e1[DE + D:]

    nb = 10
    rows = N // nb
    ps, pr = pl.pallas_call(
        _prep_body,
        grid=(nb,),
        in_specs=[
            pl.BlockSpec((rows, D), lambda i: (i, 0)),
            pl.BlockSpec((D, D), lambda i: (0, 0)),
            pl.BlockSpec((D, D), lambda i: (0, 0)),
        ],
        out_specs=[
            pl.BlockSpec((rows, D), lambda i: (i, 0)),
            pl.BlockSpec((rows, D), lambda i: (i, 0)),
        ],
        out_shape=[
            jax.ShapeDtypeStruct((N, D), jnp.float32),
            jax.ShapeDtypeStruct((N, D), jnp.float32),
        ],
    )(nodes, w_s, w_r)

    neb = 32
    erows = E // neb
    edgebase = pl.pallas_call(
        _edgebase_body,
        grid=(neb,),
        in_specs=[
            pl.BlockSpec((erows, DE), lambda i: (i, 0)),
            pl.BlockSpec((DE, D), lambda i: (0, 0)),
            pl.BlockSpec((1, D), lambda i: (0, 0)),
        ],
        out_specs=pl.BlockSpec((erows, D), lambda i: (i, 0)),
        out_shape=jax.ShapeDtypeStruct((E, D), jnp.float32),
    )(edges, w_edge, b_e1.reshape(1, D))

    mesh = plsc.VectorSubcoreMesh(core_axis_name="c", subcore_axis_name="s")
    acc2, cnt2 = pl.kernel(
        _sc_body,
        out_type=(
            jax.ShapeDtypeStruct((NC, N, D), jnp.float32),
            jax.ShapeDtypeStruct((NC, NCNT), jnp.float32),
        ),
        mesh=mesh,
        scratch_types=[
            pltpu.VMEM((EB,), jnp.int32),
            pltpu.VMEM((EB,), jnp.int32),
            pltpu.VMEM((EB, D), jnp.float32),
            pltpu.VMEM((EB, D), jnp.float32),
            pltpu.VMEM((EB, D), jnp.float32),
            pltpu.VMEM((EB, D), jnp.float32),
            pltpu.VMEM((EB,), jnp.float32),
            pltpu.VMEM((ZR, D), jnp.float32),
            pltpu.VMEM((CPAD,), jnp.float32),
            pltpu.VMEM_SHARED((N, D), jnp.float32),
            pltpu.VMEM_SHARED((NCNT,), jnp.float32),
            pltpu.SemaphoreType.DMA,
            pltpu.SemaphoreType.DMA,
        ],
    )(ps, pr, edgebase, senders, receivers)

    cnt2n = cnt2.reshape(NC, NCNT, 1)[:, :N, :]

    nb2 = 10
    rows2 = N // nb2
    out = pl.pallas_call(
        _post_body,
        grid=(nb2,),
        in_specs=[
            pl.BlockSpec((NC, rows2, D), lambda i: (0, i, 0)),
            pl.BlockSpec((NC, rows2, 1), lambda i: (0, i, 0)),
            pl.BlockSpec((rows2, D), lambda i: (i, 0)),
            pl.BlockSpec((D,), lambda i: (0,)),
            pl.BlockSpec((D,), lambda i: (0,)),
            pl.BlockSpec((D, D), lambda i: (0, 0)),
            pl.BlockSpec((D,), lambda i: (0,)),
            pl.BlockSpec((D, D), lambda i: (0, 0)),
            pl.BlockSpec((D, D), lambda i: (0, 0)),
            pl.BlockSpec((D,), lambda i: (0,)),
            pl.BlockSpec((D,), lambda i: (0,)),
            pl.BlockSpec((D,), lambda i: (0,)),
            pl.BlockSpec((D, D), lambda i: (0, 0)),
            pl.BlockSpec((D,), lambda i: (0,)),
        ],
        out_specs=pl.BlockSpec((rows2, D), lambda i: (i, 0)),
        out_shape=jax.ShapeDtypeStruct((N, D), jnp.float32),
    )(acc2, cnt2n, nodes,
      scale_e, offset_e, W_e2, b_e2,
      W_n1[:D], W_n1[D:], b_n1, scale_n, offset_n, W_n2, b_n2)
    return out
